# Initial kernel scaffold; baseline (speedup 1.0000x reference)
import jax
import jax.numpy as jnp
from jax.experimental import pallas as pl


def kernel(x, edge_index, params):
    raise NotImplementedError("write your pallas kernel here")


# speedup vs baseline: 6.5549x; 6.5549x over previous
import functools

import jax
import jax.numpy as jnp
from jax import lax
from jax.experimental import pallas as pl
from jax.experimental.pallas import tpu as pltpu
from jax.experimental.pallas import tpu_sc as plsc

DEPTH = 4
NH = 128
NF = 256
HEADS = 5
N = 10000
E = 160000
EL = E + N

SC_CORES = 2
SC_SUBCORES = 16
NW = SC_CORES * SC_SUBCORES
CHUNK = 64

N_PAD = 10240
EL_PAD = 2048 * 84
E_PAD = 2048 * 79
ETILE = 2048


@functools.lru_cache(None)
def _sc_gather(B, D):
    b_per_w = B // NW
    n_chunks = b_per_w // CHUNK
    mesh = plsc.VectorSubcoreMesh(core_axis_name="c", subcore_axis_name="s")

    @functools.partial(
        pl.kernel, mesh=mesh,
        out_type=jax.ShapeDtypeStruct((B, D), jnp.float32),
        scratch_types=[
            pltpu.VMEM((CHUNK,), jnp.int32),
            pltpu.VMEM((CHUNK, D), jnp.float32),
            pltpu.SemaphoreType.DMA,
        ],
    )
    def gk(tbl_hbm, idx_hbm, out_hbm, idx_v, rows_v, sem):
        wid = lax.axis_index("s") * SC_CORES + lax.axis_index("c")
        base = wid * b_per_w

        def body(t, carry):
            off = base + t * CHUNK
            pltpu.sync_copy(idx_hbm.at[pl.ds(off, CHUNK)], idx_v)
            pltpu.async_copy(tbl_hbm.at[idx_v], rows_v, sem).wait()
            pltpu.sync_copy(rows_v, out_hbm.at[pl.ds(off, CHUNK), :])
            return carry

        lax.fori_loop(0, n_chunks, body, 0)

    return gk


@functools.lru_cache(None)
def _sc_scatter_add(B, D):
    b_per_w = B // NW
    n_chunks = b_per_w // CHUNK
    rows_per_sub = N_PAD // SC_SUBCORES
    mesh = plsc.VectorSubcoreMesh(core_axis_name="c", subcore_axis_name="s")

    @functools.partial(
        pl.kernel, mesh=mesh,
        out_type=(jax.ShapeDtypeStruct((N_PAD, D), jnp.float32),
                  jax.ShapeDtypeStruct((N_PAD, D), jnp.float32)),
        scratch_types=[
            pltpu.VMEM((CHUNK,), jnp.int32),
            pltpu.VMEM((CHUNK, D), jnp.float32),
            pltpu.VMEM_SHARED((N_PAD, D), jnp.float32),
        ],
    )
    def sk(rows_hbm, idx_hbm, zeros_hbm, out0, out1, idx_v, rows_v, acc):
        cid = lax.axis_index("c")
        sid = lax.axis_index("s")
        wid = sid * SC_CORES + cid
        base = wid * b_per_w
        sl = pl.ds(sid * rows_per_sub, rows_per_sub)

        pltpu.sync_copy(zeros_hbm.at[sl, :], acc.at[sl, :])
        plsc.subcore_barrier()

        def body(t, carry):
            off = base + t * CHUNK
            pltpu.sync_copy(idx_hbm.at[pl.ds(off, CHUNK)], idx_v)
            pltpu.sync_copy(rows_hbm.at[pl.ds(off, CHUNK), :], rows_v)
            pltpu.sync_copy(rows_v, acc.at[idx_v], add=True)
            return carry

        lax.fori_loop(0, n_chunks, body, 0)
        plsc.subcore_barrier()

        @pl.when(cid == 0)
        def _():
            pltpu.sync_copy(acc.at[sl, :], out0.at[sl, :])

        @pl.when(cid == 1)
        def _():
            pltpu.sync_copy(acc.at[sl, :], out1.at[sl, :])

    return sk


@functools.lru_cache(None)
def _mm(nrows, k, m, tr):
    def body(x_ref, w_ref, b_ref, o_ref):
        o_ref[...] = jnp.dot(x_ref[...], w_ref[...],
                             preferred_element_type=jnp.float32) + b_ref[...]

    return pl.pallas_call(
        body,
        grid=(nrows // tr,),
        in_specs=[pl.BlockSpec((tr, k), lambda i: (i, 0)),
                  pl.BlockSpec((k, m), lambda i: (0, 0)),
                  pl.BlockSpec((1, m), lambda i: (0, 0))],
        out_specs=pl.BlockSpec((tr, m), lambda i: (i, 0)),
        out_shape=jax.ShapeDtypeStruct((nrows, m), jnp.float32),
    )


def _matmul(x, w, b=None):
    nrows, k = x.shape
    m = w.shape[1]
    if b is None:
        b = jnp.zeros((1, m), jnp.float32)
    else:
        b = b.reshape(1, m)
    tr = 1024 if nrows % 1024 == 0 else 1000
    return _mm(nrows, k, m, tr)(x, w, b)


@functools.lru_cache(None)
def _ex_kernel(B, e_real):
    def body(ga_ref, gb_ref, ex_ref, mx_ref):
        ph = pl.program_id(0)
        i = pl.program_id(1)
        a = ga_ref[...] + gb_ref[...]
        a = jnp.where(a >= 0.0, a, 0.2 * a)
        rid = (lax.broadcasted_iota(jnp.int32, a.shape, 0) + i * ETILE)
        a = jnp.where(rid < e_real, a, -jnp.inf)

        @pl.when(ph == 0)
        def _():
            @pl.when(i == 0)
            def _():
                mx_ref[...] = jnp.full(mx_ref.shape, -jnp.inf, jnp.float32)
            tmax = jnp.max(a, axis=0, keepdims=True)
            mx_ref[0:1, :] = jnp.maximum(mx_ref[0:1, :], tmax)

        @pl.when(ph == 1)
        def _():
            ex_ref[...] = jnp.exp(a - mx_ref[0:1, :])

    return pl.pallas_call(
        body,
        grid=(2, B // ETILE),
        in_specs=[pl.BlockSpec((ETILE, 8), lambda p, i: (i, 0)),
                  pl.BlockSpec((ETILE, 8), lambda p, i: (i, 0))],
        out_specs=pl.BlockSpec((ETILE, 8), lambda p, i: (i, 0)),
        out_shape=jax.ShapeDtypeStruct((B, 8), jnp.float32),
        scratch_shapes=[pltpu.VMEM((8, 8), jnp.float32)],
    )


@functools.lru_cache(None)
def _weighted_kernel(B, nheads):
    def body(r_ref, e_ref, *outs):
        r = r_ref[...]
        e = e_ref[...]
        for p in range(nheads):
            outs[p][...] = r[:, p * NH:(p + 1) * NH] * e[:, p:p + 1]

    return pl.pallas_call(
        body,
        grid=(B // ETILE,),
        in_specs=[pl.BlockSpec((ETILE, nheads * NH), lambda i: (i, 0)),
                  pl.BlockSpec((ETILE, 8), lambda i: (i, 0))],
        out_specs=[pl.BlockSpec((ETILE, NH), lambda i: (i, 0))] * nheads,
        out_shape=[jax.ShapeDtypeStruct((B, NH), jnp.float32)] * nheads,
    )


@functools.lru_cache(None)
def _combine_kernel(nheads, relu):
    def body(*refs):
        d0, d1 = refs[0], refs[1]
        b_ref = refs[2 + 2 * nheads]
        o_ref = refs[3 + 2 * nheads]
        den = d0[...][:, :8] + d1[...][:, :8]
        for p in range(nheads):
            num = refs[2 + 2 * p][...] + refs[3 + 2 * p][...]
            v = num / (den[:, p:p + 1] + 1e-16) + b_ref[:, p * NH:(p + 1) * NH]
            if relu:
                v = jnp.maximum(v, 0.0)
            o_ref[:, p * NH:(p + 1) * NH] = v

    tr = 1024
    return pl.pallas_call(
        body,
        grid=(N_PAD // tr,),
        in_specs=[pl.BlockSpec((tr, NH), lambda i: (i, 0))] * 2
        + [pl.BlockSpec((tr, NH), lambda i: (i, 0))] * (2 * nheads)
        + [pl.BlockSpec((1, nheads * NH), lambda i: (0, 0))],
        out_specs=pl.BlockSpec((tr, nheads * NH), lambda i: (i, 0)),
        out_shape=jax.ShapeDtypeStruct((N_PAD, nheads * NH), jnp.float32),
    )


@functools.lru_cache(None)
def _bn_kernel():
    def body(h_ref, s_ref, g_ref, be_ref, o_ref):
        h = h_ref[...]
        mean = jnp.mean(h, axis=0, keepdims=True)
        var = jnp.mean((h - mean) ** 2, axis=0, keepdims=True)
        xn = g_ref[...] * (h - mean) / jnp.sqrt(var + 1e-5) + be_ref[...]
        o_ref[...] = jnp.maximum(xn, 0.0) + s_ref[...]

    return pl.pallas_call(
        body, out_shape=jax.ShapeDtypeStruct((N, NH), jnp.float32))


@functools.lru_cache(None)
def _dinv_kernel():
    def body(d0_ref, d1_ref, o_ref):
        deg = d0_ref[...] + d1_ref[...]
        o_ref[...] = jnp.where(deg > 0.0, 1.0 / jnp.sqrt(deg), 0.0)

    return pl.pallas_call(
        body, out_shape=jax.ShapeDtypeStruct((N_PAD, NH), jnp.float32))


@functools.lru_cache(None)
def _cheb_weighted_kernel():
    npanels = 3

    def body(r_ref, a_ref, b_ref, *outs):
        w = -(a_ref[...][:, :1] * b_ref[...][:, :1])
        rid = (lax.broadcasted_iota(jnp.int32, (ETILE, 1), 0)
               + pl.program_id(0) * ETILE)
        w = jnp.where(rid < E, w, 0.0)
        r = r_ref[...]
        for p in range(npanels):
            outs[p][...] = r[:, p * NH:(p + 1) * NH] * w

    return pl.pallas_call(
        body,
        grid=(E_PAD // ETILE,),
        in_specs=[pl.BlockSpec((ETILE, npanels * NH), lambda i: (i, 0)),
                  pl.BlockSpec((ETILE, NH), lambda i: (i, 0)),
                  pl.BlockSpec((ETILE, NH), lambda i: (i, 0))],
        out_specs=[pl.BlockSpec((ETILE, NH), lambda i: (i, 0))] * npanels,
        out_shape=[jax.ShapeDtypeStruct((E_PAD, NH), jnp.float32)] * npanels,
    )


@functools.lru_cache(None)
def _sum_partials_kernel(npanels):
    def body(*refs):
        o_ref = refs[2 * npanels]
        for p in range(npanels):
            o_ref[:, p * NH:(p + 1) * NH] = refs[2 * p][...] + refs[2 * p + 1][...]

    tr = 1024
    return pl.pallas_call(
        body,
        grid=(N_PAD // tr,),
        in_specs=[pl.BlockSpec((tr, NH), lambda i: (i, 0))] * (2 * npanels),
        out_specs=pl.BlockSpec((tr, npanels * NH), lambda i: (i, 0)),
        out_shape=jax.ShapeDtypeStruct((N_PAD, npanels * NH), jnp.float32),
    )


def _att_matrices(att_src, att_dst):
    heads, ch = att_src.shape
    eye = jnp.eye(heads, 8, dtype=jnp.float32)
    a_s = (att_src[:, :, None] * eye[:, None, :]).reshape(heads * ch, 8)
    a_d = (att_dst[:, :, None] * eye[:, None, :]).reshape(heads * ch, 8)
    return a_s, a_d


def _gat_conv(h, src_p, dst_p, att_src, att_dst, bias, heads, zeros128,
              relu_out):
    a_s, a_d = _att_matrices(att_src, att_dst)
    rows = _sc_gather(EL_PAD, heads * NH)(h, src_p)
    ga = _matmul(rows, a_s)
    sdst = _matmul(h, a_d)
    sdst128 = jnp.pad(sdst, ((0, 0), (0, NH - 8)))
    gb = _sc_gather(EL_PAD, NH)(sdst128, dst_p)
    ex = _ex_kernel(EL_PAD, EL)(ga, gb[:, :8])
    ex128 = jnp.pad(ex, ((0, 0), (0, NH - 8)))
    den0, den1 = _sc_scatter_add(EL_PAD, NH)(ex128, dst_p, zeros128)

    wpanels = _weighted_kernel(EL_PAD, heads)(rows, ex)
    partials = []
    for p in range(heads):
        p0, p1 = _sc_scatter_add(EL_PAD, NH)(wpanels[p], dst_p, zeros128)
        partials += [p0, p1]
    out = _combine_kernel(heads, relu_out)(
        den0, den1, *partials, bias.reshape(1, heads * NH))
    return out[:N]


def kernel(x, edge_index, params):
    src = edge_index[0].astype(jnp.int32)
    dst = edge_index[1].astype(jnp.int32)
    loop = jnp.arange(N, dtype=jnp.int32)
    padl = jnp.zeros((EL_PAD - EL,), jnp.int32)
    src_p = jnp.concatenate([src, loop, padl])
    dst_p = jnp.concatenate([dst, loop, padl])
    pade = jnp.zeros((E_PAD - E,), jnp.int32)
    row_p = jnp.concatenate([src, pade])
    col_p = jnp.concatenate([dst, pade])

    zeros128 = jnp.zeros((N_PAD, NH), jnp.float32)
    ones128 = jnp.zeros((E_PAD, NH), jnp.float32).at[:E].set(1.0)

    x0 = x
    for i in range(DEPTH):
        p = params['block%d' % i]
        h1 = _matmul(x, p['W1'])
        h1a = _gat_conv(h1, src_p, dst_p, p['as1'], p['ad1'], p['b1'],
                        HEADS, zeros128, relu_out=True)
        h2 = _matmul(h1a, p['W2'])
        h2o = _gat_conv(h2, src_p, dst_p, p['as2'], p['ad2'], p['b2'],
                        1, zeros128, relu_out=False)
        skip = _matmul(x, p['Wskip'], p['bskip'])
        x = _bn_kernel()(h2o, skip, p['gamma'].reshape(1, NH),
                         p['beta'].reshape(1, NH))

    xm = jnp.concatenate([x, x0], axis=1)

    d0, d1 = _sc_scatter_add(E_PAD, NH)(ones128, row_p, zeros128)
    dinv128 = _dinv_kernel()(d0, d1)
    gr = _sc_gather(E_PAD, NH)(dinv128, row_p)
    gc = _sc_gather(E_PAD, NH)(dinv128, col_p)
    xm_rows = _sc_gather(E_PAD, 3 * NH)(xm, row_p)
    w0, w1, w2 = _cheb_weighted_kernel()(xm_rows, gr, gc)
    tparts = []
    for wp in (w0, w1, w2):
        t0, t1 = _sc_scatter_add(E_PAD, NH)(wp, col_p, zeros128)
        tparts += [t0, t1]
    tx1 = _sum_partials_kernel(3)(*tparts)[:N]

    pm = params['mix']
    xcat = jnp.concatenate([xm, tx1], axis=1)
    wcat = jnp.concatenate([pm['W0'], pm['W1']], axis=0)
    return _matmul(xcat, wcat, pm['b'])


# speedup vs baseline: 7.3558x; 1.1222x over previous
import functools

import jax
import jax.numpy as jnp
from jax import lax
from jax.experimental import pallas as pl
from jax.experimental.pallas import tpu as pltpu
from jax.experimental.pallas import tpu_sc as plsc

DEPTH = 4
NH = 128
HEADS = 5
N = 10000
E = 160000
EL = E + N

SC_CORES = 2
SC_SUBCORES = 16
NW = SC_CORES * SC_SUBCORES
CHUNK = 128

N_PAD = 10240
EL_PAD = 2048 * 84
E_PAD = 2048 * 80
ETILE = 2048


@functools.lru_cache(None)
def _sc_gather(B, D):
    b_per_w = B // NW
    n_chunks = b_per_w // CHUNK
    mesh = plsc.VectorSubcoreMesh(core_axis_name="c", subcore_axis_name="s")

    @functools.partial(
        pl.kernel, mesh=mesh,
        out_type=jax.ShapeDtypeStruct((B, D), jnp.float32),
        scratch_types=[
            pltpu.VMEM((CHUNK,), jnp.int32),
            pltpu.VMEM((CHUNK, D), jnp.float32),
            pltpu.SemaphoreType.DMA,
        ],
    )
    def gk(tbl_hbm, idx_hbm, out_hbm, idx_v, rows_v, sem):
        wid = lax.axis_index("s") * SC_CORES + lax.axis_index("c")
        base = wid * b_per_w

        def body(t, carry):
            off = base + t * CHUNK
            pltpu.sync_copy(idx_hbm.at[pl.ds(off, CHUNK)], idx_v)
            pltpu.async_copy(tbl_hbm.at[idx_v], rows_v, sem).wait()
            pltpu.sync_copy(rows_v, out_hbm.at[pl.ds(off, CHUNK), :])
            return carry

        lax.fori_loop(0, n_chunks, body, 0)

    return gk


@functools.lru_cache(None)
def _sc_agg(B, ntbl, npanels, denom):
    b_per_w = B // NW
    n_chunks = b_per_w // CHUNK
    rows_per_sub = N_PAD // SC_SUBCORES
    n_ex = 1 if (npanels == 0) else npanels
    n_pass = npanels + (1 if denom else 0)
    mesh = plsc.VectorSubcoreMesh(core_axis_name="c", subcore_axis_name="s")

    scratch = [pltpu.VMEM((CHUNK,), jnp.int32),
               pltpu.VMEM((CHUNK,), jnp.int32),
               pltpu.VMEM((CHUNK, NH), jnp.float32),
               pltpu.VMEM_SHARED((N_PAD, NH), jnp.float32),
               pltpu.SemaphoreType.DMA]
    if denom:
        scratch.insert(3, pltpu.VMEM((CHUNK, NH), jnp.float32))
    scratch = [pltpu.VMEM((CHUNK,), jnp.float32)] * n_ex + scratch

    @functools.partial(
        pl.kernel, mesh=mesh,
        out_type=tuple(jax.ShapeDtypeStruct((N_PAD, NH), jnp.float32)
                       for _ in range(2 * n_pass)),
        scratch_types=scratch,
    )
    def ak(*refs):
        pos = 0
        tbls = refs[pos:pos + npanels]; pos += npanels
        ex_hbm = refs[pos:pos + n_ex]; pos += n_ex
        src_hbm = refs[pos]; pos += 1
        dst_hbm = refs[pos]; pos += 1
        zeros_hbm = refs[pos]; pos += 1
        outs = refs[pos:pos + 2 * n_pass]; pos += 2 * n_pass
        ex_v = refs[pos:pos + n_ex]; pos += n_ex
        sidx_v = refs[pos]; pos += 1
        didx_v = refs[pos]; pos += 1
        rows_v = refs[pos]; pos += 1
        if denom:
            den_v = refs[pos]; pos += 1
        acc = refs[pos]; pos += 1
        sem = refs[pos]

        cid = lax.axis_index("c")
        sid = lax.axis_index("s")
        wid = sid * SC_CORES + cid
        base = wid * b_per_w
        sl = pl.ds(sid * rows_per_sub, rows_per_sub)

        if denom:
            def zden(r, carry):
                for j in range(NH // 16):
                    den_v[r, pl.ds(16 * j, 16)] = jnp.zeros((16,), jnp.float32)
                return carry
            lax.fori_loop(0, CHUNK, zden, 0)
        lanes = lax.iota(jnp.int32, 16)

        for ps in range(n_pass):
            is_den = ps == npanels
            pltpu.sync_copy(zeros_hbm.at[sl, :], acc.at[sl, :])
            plsc.subcore_barrier()

            def body(t, carry):
                off = base + t * CHUNK
                pltpu.sync_copy(dst_hbm.at[pl.ds(off, CHUNK)], didx_v)
                if is_den:
                    for j in range(n_ex):
                        pltpu.sync_copy(ex_hbm[j].at[pl.ds(off, CHUNK)], ex_v[j])
                else:
                    pltpu.sync_copy(ex_hbm[ps].at[pl.ds(off, CHUNK)], ex_v[ps])
                    pltpu.sync_copy(src_hbm.at[pl.ds(off, CHUNK)], sidx_v)
                    pltpu.async_copy(tbls[ps].at[sidx_v], rows_v, sem).wait()

                if is_den:
                    def dbody(g, c2):
                        evs = [ex_v[j][pl.ds(g * 16, 16)] for j in range(n_ex)]
                        for rr in range(16):
                            v = jnp.zeros((16,), jnp.float32)
                            for j in range(n_ex):
                                v = v + jnp.where(lanes == j, evs[j][rr], 0.0)
                            den_v[g * 16 + rr, pl.ds(0, 16)] = v
                        return c2

                    lax.fori_loop(0, CHUNK // 16, dbody, 0)
                    pltpu.sync_copy(den_v, acc.at[didx_v], add=True)
                else:
                    def sbody(g, c2):
                        wv = ex_v[ps][pl.ds(g * 16, 16)]
                        for rr in range(16):
                            w = wv[rr]
                            r = g * 16 + rr
                            for j in range(NH // 16):
                                d = pl.ds(16 * j, 16)
                                rows_v[r, d] = rows_v[r, d] * w
                        return c2

                    lax.fori_loop(0, CHUNK // 16, sbody, 0)
                    pltpu.sync_copy(rows_v, acc.at[didx_v], add=True)
                return carry

            lax.fori_loop(0, n_chunks, body, 0)
            plsc.subcore_barrier()

            @pl.when(cid == 0)
            def _():
                pltpu.sync_copy(acc.at[sl, :], outs[2 * ps].at[sl, :])

            @pl.when(cid == 1)
            def _():
                pltpu.sync_copy(acc.at[sl, :], outs[2 * ps + 1].at[sl, :])

    return ak


@functools.lru_cache(None)
def _mm(nrows, k, m, tr, prec):
    def body(x_ref, w_ref, b_ref, o_ref):
        o_ref[...] = jnp.dot(x_ref[...], w_ref[...], precision=prec,
                             preferred_element_type=jnp.float32) + b_ref[...]

    return pl.pallas_call(
        body,
        grid=(nrows // tr,),
        in_specs=[pl.BlockSpec((tr, k), lambda i: (i, 0)),
                  pl.BlockSpec((k, m), lambda i: (0, 0)),
                  pl.BlockSpec((1, m), lambda i: (0, 0))],
        out_specs=pl.BlockSpec((tr, m), lambda i: (i, 0)),
        out_shape=jax.ShapeDtypeStruct((nrows, m), jnp.float32),
    )


def _matmul(x, w, b=None, exact=False):
    nrows, k = x.shape
    m = w.shape[1]
    if b is None:
        b = jnp.zeros((1, m), jnp.float32)
    else:
        b = b.reshape(1, m)
    tr = 1024 if nrows % 1024 == 0 else 1000
    prec = lax.Precision.HIGHEST if exact else lax.Precision.DEFAULT
    return _mm(nrows, k, m, tr, prec)(x, w, b)


@functools.lru_cache(None)
def _ex_kernel(B, e_real):
    def body(ga_ref, gb_ref, ex_ref, mx_ref):
        ph = pl.program_id(0)
        i = pl.program_id(1)
        a = ga_ref[...][:, :8] + gb_ref[...][:, :8]
        a = jnp.where(a >= 0.0, a, 0.2 * a)
        rid = (lax.broadcasted_iota(jnp.int32, a.shape, 0) + i * ETILE)
        a = jnp.where(rid < e_real, a, -jnp.inf)

        @pl.when(ph == 0)
        def _():
            @pl.when(i == 0)
            def _():
                mx_ref[...] = jnp.full(mx_ref.shape, -jnp.inf, jnp.float32)
            tmax = jnp.max(a, axis=0, keepdims=True)
            mx_ref[0:1, :] = jnp.maximum(mx_ref[0:1, :], tmax)

        @pl.when(ph == 1)
        def _():
            e = jnp.exp(a - mx_ref[0:1, :])
            ex_ref[...] = jnp.transpose(e)

    return pl.pallas_call(
        body,
        grid=(2, B // ETILE),
        in_specs=[pl.BlockSpec((ETILE, NH), lambda p, i: (i, 0)),
                  pl.BlockSpec((ETILE, NH), lambda p, i: (i, 0))],
        out_specs=pl.BlockSpec((8, ETILE), lambda p, i: (0, i)),
        out_shape=jax.ShapeDtypeStruct((8, B), jnp.float32),
        scratch_shapes=[pltpu.VMEM((8, 8), jnp.float32)],
    )


@functools.lru_cache(None)
def _combine_kernel(nheads, relu):
    def body(*refs):
        d0, d1 = refs[0], refs[1]
        b_ref = refs[2 + 2 * nheads]
        o_ref = refs[3 + 2 * nheads]
        den = d0[...][:, :8] + d1[...][:, :8]
        for p in range(nheads):
            num = refs[2 + 2 * p][...] + refs[3 + 2 * p][...]
            v = num / (den[:, p:p + 1] + 1e-16) + b_ref[:, p * NH:(p + 1) * NH]
            if relu:
                v = jnp.maximum(v, 0.0)
            o_ref[:, p * NH:(p + 1) * NH] = v

    tr = 1024
    return pl.pallas_call(
        body,
        grid=(N_PAD // tr,),
        in_specs=[pl.BlockSpec((tr, NH), lambda i: (i, 0))] * 2
        + [pl.BlockSpec((tr, NH), lambda i: (i, 0))] * (2 * nheads)
        + [pl.BlockSpec((1, nheads * NH), lambda i: (0, 0))],
        out_specs=pl.BlockSpec((tr, nheads * NH), lambda i: (i, 0)),
        out_shape=jax.ShapeDtypeStruct((N_PAD, nheads * NH), jnp.float32),
    )


@functools.lru_cache(None)
def _bn_kernel():
    def body(h_ref, s_ref, g_ref, be_ref, o_ref):
        h = h_ref[...]
        mean = jnp.mean(h, axis=0, keepdims=True)
        var = jnp.mean((h - mean) ** 2, axis=0, keepdims=True)
        xn = g_ref[...] * (h - mean) / jnp.sqrt(var + 1e-5) + be_ref[...]
        o_ref[...] = jnp.maximum(xn, 0.0) + s_ref[...]

    return pl.pallas_call(
        body, out_shape=jax.ShapeDtypeStruct((N, NH), jnp.float32))


@functools.lru_cache(None)
def _dinv_kernel():
    def body(d0_ref, d1_ref, o_ref):
        deg = d0_ref[...] + d1_ref[...]
        o_ref[...] = jnp.where(deg > 0.0, 1.0 / jnp.sqrt(deg), 0.0)

    return pl.pallas_call(
        body, out_shape=jax.ShapeDtypeStruct((N_PAD, NH), jnp.float32))


@functools.lru_cache(None)
def _chebw_kernel():
    def body(a_ref, b_ref, o_ref):
        w = -(a_ref[...][:, 0] * b_ref[...][:, 0])
        rid = (lax.iota(jnp.int32, ETILE) + pl.program_id(0) * ETILE)
        o_ref[...] = jnp.where(rid < E, w, 0.0)

    return pl.pallas_call(
        body,
        grid=(E_PAD // ETILE,),
        in_specs=[pl.BlockSpec((ETILE, NH), lambda i: (i, 0)),
                  pl.BlockSpec((ETILE, NH), lambda i: (i, 0))],
        out_specs=pl.BlockSpec((ETILE,), lambda i: (i,)),
        out_shape=jax.ShapeDtypeStruct((E_PAD,), jnp.float32),
    )


@functools.lru_cache(None)
def _sum_partials_kernel(npanels):
    def body(*refs):
        o_ref = refs[2 * npanels]
        for p in range(npanels):
            o_ref[:, p * NH:(p + 1) * NH] = refs[2 * p][...] + refs[2 * p + 1][...]

    tr = 1024
    return pl.pallas_call(
        body,
        grid=(N_PAD // tr,),
        in_specs=[pl.BlockSpec((tr, NH), lambda i: (i, 0))] * (2 * npanels),
        out_specs=pl.BlockSpec((tr, npanels * NH), lambda i: (i, 0)),
        out_shape=jax.ShapeDtypeStruct((N_PAD, npanels * NH), jnp.float32),
    )


def _att_matrices(att_src, att_dst):
    heads, ch = att_src.shape
    eye = jnp.eye(heads, 8, dtype=jnp.float32)
    a_s = (att_src[:, :, None] * eye[:, None, :]).reshape(heads * ch, 8)
    a_d = (att_dst[:, :, None] * eye[:, None, :]).reshape(heads * ch, 8)
    return a_s, a_d


def _gat_conv(h, src_p, dst_p, att_src, att_dst, bias, heads, zeros128,
              relu_out):
    a_s, a_d = _att_matrices(att_src, att_dst)
    sa = jnp.pad(_matmul(h, a_s, exact=True), ((0, 0), (0, NH - 8)))
    sd = jnp.pad(_matmul(h, a_d, exact=True), ((0, 0), (0, NH - 8)))
    ga = _sc_gather(EL_PAD, NH)(sa, src_p)
    gb = _sc_gather(EL_PAD, NH)(sd, dst_p)
    exT = _ex_kernel(EL_PAD, EL)(ga, gb)

    tables = [h[:, p * NH:(p + 1) * NH] for p in range(heads)]
    exs = [exT[p] for p in range(heads)]
    parts = _sc_agg(EL_PAD, N, heads, True)(
        *tables, *exs, src_p, dst_p, zeros128)
    den0, den1 = parts[2 * heads], parts[2 * heads + 1]
    out = _combine_kernel(heads, relu_out)(
        den0, den1, *parts[:2 * heads], bias.reshape(1, heads * NH))
    return out[:N]


def kernel(x, edge_index, params):
    src = edge_index[0].astype(jnp.int32)
    dst = edge_index[1].astype(jnp.int32)
    loop = jnp.arange(N, dtype=jnp.int32)
    padl = jnp.zeros((EL_PAD - EL,), jnp.int32)
    src_p = jnp.concatenate([src, loop, padl])
    dst_p = jnp.concatenate([dst, loop, padl])
    pade = jnp.full((E_PAD - E,), N, jnp.int32)
    row_p = jnp.concatenate([src, pade])
    col_p = jnp.concatenate([dst, pade])

    zeros128 = jnp.zeros((N_PAD, NH), jnp.float32)
    ones_e = jnp.concatenate([jnp.ones((E,), jnp.float32),
                              jnp.zeros((E_PAD - E,), jnp.float32)])

    x0 = x
    for i in range(DEPTH):
        p = params['block%d' % i]
        h1 = _matmul(x, p['W1'])
        h1a = _gat_conv(h1, src_p, dst_p, p['as1'], p['ad1'], p['b1'],
                        HEADS, zeros128, relu_out=True)
        h2 = _matmul(h1a, p['W2'])
        h2o = _gat_conv(h2, src_p, dst_p, p['as2'], p['ad2'], p['b2'],
                        1, zeros128, relu_out=False)
        skip = _matmul(x, p['Wskip'], p['bskip'])
        x = _bn_kernel()(h2o, skip, p['gamma'].reshape(1, NH),
                         p['beta'].reshape(1, NH))

    xm = jnp.concatenate([x, x0], axis=1)

    dparts = _sc_agg(E_PAD, N, 0, True)(ones_e, row_p, row_p, zeros128)
    dinv128 = _dinv_kernel()(dparts[0], dparts[1])
    gr = _sc_gather(E_PAD, NH)(dinv128, row_p)
    gc = _sc_gather(E_PAD, NH)(dinv128, col_p)
    w = _chebw_kernel()(gr, gc)

    xm_pad = jnp.pad(xm, ((0, N_PAD - N), (0, 0)))
    xtables = [xm_pad[:, p * NH:(p + 1) * NH] for p in range(3)]
    tparts = _sc_agg(E_PAD, N_PAD, 3, False)(
        *xtables, w, w, w, row_p, col_p, zeros128)
    tx1 = _sum_partials_kernel(3)(*tparts)[:N]

    pm = params['mix']
    xcat = jnp.concatenate([xm, tx1], axis=1)
    wcat = jnp.concatenate([pm['W0'], pm['W1']], axis=0)
    return _matmul(xcat, wcat, pm['b'])


# speedup vs baseline: 9.0244x; 1.2268x over previous
import functools

import jax
import jax.numpy as jnp
from jax import lax
from jax.experimental import pallas as pl
from jax.experimental.pallas import tpu as pltpu
from jax.experimental.pallas import tpu_sc as plsc

DEPTH = 4
NH = 128
HEADS = 5
N = 10000
E = 160000
EL = E + N

SC_CORES = 2
SC_SUBCORES = 16
NW = SC_CORES * SC_SUBCORES
CHUNK = 128

N_PAD = 10240
EL_PAD = 2048 * 84
E_PAD = 2048 * 80
ETILE = 2048


@functools.lru_cache(None)
def _sc_gather(B, D):
    b_per_w = B // NW
    n_chunks = b_per_w // CHUNK
    mesh = plsc.VectorSubcoreMesh(core_axis_name="c", subcore_axis_name="s")

    @functools.partial(
        pl.kernel, mesh=mesh,
        out_type=jax.ShapeDtypeStruct((B, D), jnp.float32),
        scratch_types=[
            pltpu.VMEM((CHUNK,), jnp.int32),
            pltpu.VMEM((CHUNK, D), jnp.float32),
            pltpu.SemaphoreType.DMA,
        ],
    )
    def gk(tbl_hbm, idx_hbm, out_hbm, idx_v, rows_v, sem):
        wid = lax.axis_index("s") * SC_CORES + lax.axis_index("c")
        base = wid * b_per_w

        def body(t, carry):
            off = base + t * CHUNK
            pltpu.sync_copy(idx_hbm.at[pl.ds(off, CHUNK)], idx_v)
            pltpu.async_copy(tbl_hbm.at[idx_v], rows_v, sem).wait()
            pltpu.sync_copy(rows_v, out_hbm.at[pl.ds(off, CHUNK), :])
            return carry

        lax.fori_loop(0, n_chunks, body, 0)

    return gk


@functools.lru_cache(None)
def _sc_gather2(B, D):
    b_per_w = B // NW
    gc_ = 256 if b_per_w % 256 == 0 else 128
    n_chunks = b_per_w // gc_
    mesh = plsc.VectorSubcoreMesh(core_axis_name="c", subcore_axis_name="s")

    @functools.partial(
        pl.kernel, mesh=mesh,
        out_type=(jax.ShapeDtypeStruct((B, D), jnp.float32),
                  jax.ShapeDtypeStruct((B, D), jnp.float32)),
        scratch_types=[
            pltpu.VMEM((gc_,), jnp.int32),
            pltpu.VMEM((gc_,), jnp.int32),
            pltpu.VMEM((gc_, D), jnp.float32),
            pltpu.VMEM((gc_, D), jnp.float32),
            pltpu.SemaphoreType.DMA,
            pltpu.SemaphoreType.DMA,
            pltpu.SemaphoreType.DMA,
            pltpu.SemaphoreType.DMA,
        ],
    )
    def gk(t1, t2, i1, i2, o1, o2, i1v, i2v, r1v, r2v, s1, s2, s3, s4):
        wid = lax.axis_index("s") * SC_CORES + lax.axis_index("c")
        base = wid * b_per_w

        def body(t, carry):
            off = base + t * gc_
            sl = pl.ds(off, gc_)
            c1 = pltpu.async_copy(i1.at[sl], i1v, s1)
            c2 = pltpu.async_copy(i2.at[sl], i2v, s2)
            c1.wait()
            g1 = pltpu.async_copy(t1.at[i1v], r1v, s3)
            c2.wait()
            g2 = pltpu.async_copy(t2.at[i2v], r2v, s4)
            g1.wait()
            w1 = pltpu.async_copy(r1v, o1.at[sl, :], s1)
            g2.wait()
            w2 = pltpu.async_copy(r2v, o2.at[sl, :], s2)
            w1.wait()
            w2.wait()
            return carry

        lax.fori_loop(0, n_chunks, body, 0)

    return gk


@functools.lru_cache(None)
def _sc_agg(B, ntbl, npanels, denom, ck):
    b_per_w = B // NW
    n_chunks = b_per_w // ck
    rows_per_sub = N_PAD // SC_SUBCORES
    n_ex = 1 if (npanels == 0) else npanels
    n_pass = npanels + (1 if denom else 0)
    mesh = plsc.VectorSubcoreMesh(core_axis_name="c", subcore_axis_name="s")

    scratch = [pltpu.VMEM((ck,), jnp.int32),
               pltpu.VMEM((ck,), jnp.int32),
               pltpu.VMEM((ck, NH), jnp.float32),
               pltpu.VMEM_SHARED((N_PAD, NH), jnp.float32),
               pltpu.SemaphoreType.DMA,
               pltpu.SemaphoreType.DMA,
               pltpu.SemaphoreType.DMA,
               pltpu.SemaphoreType.DMA]
    if denom:
        scratch.insert(3, pltpu.VMEM((ck, NH), jnp.float32))
    scratch = [pltpu.VMEM((ck,), jnp.float32)] * n_ex + scratch

    @functools.partial(
        pl.kernel, mesh=mesh,
        out_type=tuple(jax.ShapeDtypeStruct((N_PAD, NH), jnp.float32)
                       for _ in range(2 * n_pass)),
        scratch_types=scratch,
    )
    def ak(*refs):
        pos = 0
        tbls = refs[pos:pos + npanels]; pos += npanels
        ex_hbm = refs[pos:pos + n_ex]; pos += n_ex
        src_hbm = refs[pos]; pos += 1
        dst_hbm = refs[pos]; pos += 1
        zeros_hbm = refs[pos]; pos += 1
        outs = refs[pos:pos + 2 * n_pass]; pos += 2 * n_pass
        ex_v = refs[pos:pos + n_ex]; pos += n_ex
        sidx_v = refs[pos]; pos += 1
        didx_v = refs[pos]; pos += 1
        rows_v = refs[pos]; pos += 1
        if denom:
            den_v = refs[pos]; pos += 1
        acc = refs[pos]; pos += 1
        sem_d, sem_e, sem_s, sem_g = refs[pos:pos + 4]

        cid = lax.axis_index("c")
        sid = lax.axis_index("s")
        wid = sid * SC_CORES + cid
        base = wid * b_per_w
        sl = pl.ds(sid * rows_per_sub, rows_per_sub)

        if denom:
            def zden(r, carry):
                for j in range(NH // 16):
                    den_v[r, pl.ds(16 * j, 16)] = jnp.zeros((16,), jnp.float32)
                return carry
            lax.fori_loop(0, ck, zden, 0)
        lanes = lax.iota(jnp.int32, 16)

        for ps in range(n_pass):
            is_den = ps == npanels
            pltpu.sync_copy(zeros_hbm.at[sl, :], acc.at[sl, :])
            plsc.subcore_barrier()

            def body(t, carry):
                off = base + t * ck
                osl = pl.ds(off, ck)
                cd = pltpu.async_copy(dst_hbm.at[osl], didx_v, sem_d)
                if is_den:
                    ws = [pltpu.async_copy(ex_hbm[j].at[osl], ex_v[j], sem_e)
                          for j in range(n_ex)]
                    for wcp in ws:
                        wcp.wait()
                else:
                    ce = pltpu.async_copy(ex_hbm[ps].at[osl], ex_v[ps], sem_e)
                    cs = pltpu.async_copy(src_hbm.at[osl], sidx_v, sem_s)
                    cs.wait()
                    cg = pltpu.async_copy(tbls[ps].at[sidx_v], rows_v, sem_g)
                    ce.wait()
                    cg.wait()

                if is_den:
                    def dbody(g, c2):
                        evs = [ex_v[j][pl.ds(g * 16, 16)] for j in range(n_ex)]
                        for rr in range(16):
                            v = jnp.zeros((16,), jnp.float32)
                            for j in range(n_ex):
                                v = v + jnp.where(lanes == j, evs[j][rr], 0.0)
                            den_v[g * 16 + rr, pl.ds(0, 16)] = v
                        return c2

                    lax.fori_loop(0, ck // 16, dbody, 0)
                    cd.wait()
                    pltpu.sync_copy(den_v, acc.at[didx_v], add=True)
                else:
                    def sbody(g, c2):
                        wv = ex_v[ps][pl.ds(g * 16, 16)]
                        for rr in range(16):
                            w = wv[rr]
                            r = g * 16 + rr
                            for j in range(NH // 16):
                                d = pl.ds(16 * j, 16)
                                rows_v[r, d] = rows_v[r, d] * w
                        return c2

                    lax.fori_loop(0, ck // 16, sbody, 0)
                    cd.wait()
                    pltpu.sync_copy(rows_v, acc.at[didx_v], add=True)
                return carry

            lax.fori_loop(0, n_chunks, body, 0)
            plsc.subcore_barrier()

            @pl.when(cid == 0)
            def _():
                pltpu.sync_copy(acc.at[sl, :], outs[2 * ps].at[sl, :])

            @pl.when(cid == 1)
            def _():
                pltpu.sync_copy(acc.at[sl, :], outs[2 * ps + 1].at[sl, :])

    return ak


@functools.lru_cache(None)
def _mm(nrows, k, m, tr, prec):
    def body(x_ref, w_ref, b_ref, o_ref):
        o_ref[...] = jnp.dot(x_ref[...], w_ref[...], precision=prec,
                             preferred_element_type=jnp.float32) + b_ref[...]

    return pl.pallas_call(
        body,
        grid=(nrows // tr,),
        in_specs=[pl.BlockSpec((tr, k), lambda i: (i, 0)),
                  pl.BlockSpec((k, m), lambda i: (0, 0)),
                  pl.BlockSpec((1, m), lambda i: (0, 0))],
        out_specs=pl.BlockSpec((tr, m), lambda i: (i, 0)),
        out_shape=jax.ShapeDtypeStruct((nrows, m), jnp.float32),
    )


def _matmul(x, w, b=None, exact=False):
    nrows, k = x.shape
    m = w.shape[1]
    if b is None:
        b = jnp.zeros((1, m), jnp.float32)
    else:
        b = b.reshape(1, m)
    tr = 1024 if nrows % 1024 == 0 else 1000
    prec = lax.Precision.HIGHEST if exact else lax.Precision.DEFAULT
    return _mm(nrows, k, m, tr, prec)(x, w, b)


@functools.lru_cache(None)
def _ex_kernel(B, e_real):
    def body(ga_ref, gb_ref, ex_ref, mx_ref):
        ph = pl.program_id(0)
        i = pl.program_id(1)
        a = ga_ref[...][:, :8] + gb_ref[...][:, :8]
        a = jnp.where(a >= 0.0, a, 0.2 * a)
        rid = (lax.broadcasted_iota(jnp.int32, a.shape, 0) + i * ETILE)
        a = jnp.where(rid < e_real, a, -jnp.inf)

        @pl.when(ph == 0)
        def _():
            @pl.when(i == 0)
            def _():
                mx_ref[...] = jnp.full(mx_ref.shape, -jnp.inf, jnp.float32)
            tmax = jnp.max(a, axis=0, keepdims=True)
            mx_ref[0:1, :] = jnp.maximum(mx_ref[0:1, :], tmax)

        @pl.when(ph == 1)
        def _():
            e = jnp.exp(a - mx_ref[0:1, :])
            ex_ref[...] = jnp.transpose(e)

    return pl.pallas_call(
        body,
        grid=(2, B // ETILE),
        in_specs=[pl.BlockSpec((ETILE, NH), lambda p, i: (i, 0)),
                  pl.BlockSpec((ETILE, NH), lambda p, i: (i, 0))],
        out_specs=pl.BlockSpec((8, ETILE), lambda p, i: (0, i)),
        out_shape=jax.ShapeDtypeStruct((8, B), jnp.float32),
        scratch_shapes=[pltpu.VMEM((8, 8), jnp.float32)],
    )


@functools.lru_cache(None)
def _combine_kernel(nheads, relu):
    def body(*refs):
        d0, d1 = refs[0], refs[1]
        b_ref = refs[2 + 2 * nheads]
        o_ref = refs[3 + 2 * nheads]
        den = d0[...][:, :8] + d1[...][:, :8]
        for p in range(nheads):
            num = refs[2 + 2 * p][...] + refs[3 + 2 * p][...]
            v = num / (den[:, p:p + 1] + 1e-16) + b_ref[:, p * NH:(p + 1) * NH]
            if relu:
                v = jnp.maximum(v, 0.0)
            o_ref[:, p * NH:(p + 1) * NH] = v

    tr = 1024
    return pl.pallas_call(
        body,
        grid=(N_PAD // tr,),
        in_specs=[pl.BlockSpec((tr, NH), lambda i: (i, 0))] * 2
        + [pl.BlockSpec((tr, NH), lambda i: (i, 0))] * (2 * nheads)
        + [pl.BlockSpec((1, nheads * NH), lambda i: (0, 0))],
        out_specs=pl.BlockSpec((tr, nheads * NH), lambda i: (i, 0)),
        out_shape=jax.ShapeDtypeStruct((N_PAD, nheads * NH), jnp.float32),
    )


@functools.lru_cache(None)
def _bn_kernel():
    def body(h_ref, s_ref, g_ref, be_ref, o_ref):
        h = h_ref[...]
        mean = jnp.mean(h, axis=0, keepdims=True)
        var = jnp.mean((h - mean) ** 2, axis=0, keepdims=True)
        xn = g_ref[...] * (h - mean) / jnp.sqrt(var + 1e-5) + be_ref[...]
        o_ref[...] = jnp.maximum(xn, 0.0) + s_ref[...]

    return pl.pallas_call(
        body, out_shape=jax.ShapeDtypeStruct((N, NH), jnp.float32))


@functools.lru_cache(None)
def _dinv_kernel():
    def body(d0_ref, d1_ref, o_ref):
        deg = d0_ref[...] + d1_ref[...]
        o_ref[...] = jnp.where(deg > 0.0, 1.0 / jnp.sqrt(deg), 0.0)

    return pl.pallas_call(
        body, out_shape=jax.ShapeDtypeStruct((N_PAD, NH), jnp.float32))


@functools.lru_cache(None)
def _chebw_kernel():
    def body(a_ref, b_ref, o_ref):
        w = -(a_ref[...][:, 0] * b_ref[...][:, 0])
        rid = (lax.iota(jnp.int32, ETILE) + pl.program_id(0) * ETILE)
        o_ref[...] = jnp.where(rid < E, w, 0.0)

    return pl.pallas_call(
        body,
        grid=(E_PAD // ETILE,),
        in_specs=[pl.BlockSpec((ETILE, NH), lambda i: (i, 0)),
                  pl.BlockSpec((ETILE, NH), lambda i: (i, 0))],
        out_specs=pl.BlockSpec((ETILE,), lambda i: (i,)),
        out_shape=jax.ShapeDtypeStruct((E_PAD,), jnp.float32),
    )


@functools.lru_cache(None)
def _sum_partials_kernel(npanels):
    def body(*refs):
        o_ref = refs[2 * npanels]
        for p in range(npanels):
            o_ref[:, p * NH:(p + 1) * NH] = refs[2 * p][...] + refs[2 * p + 1][...]

    tr = 1024
    return pl.pallas_call(
        body,
        grid=(N_PAD // tr,),
        in_specs=[pl.BlockSpec((tr, NH), lambda i: (i, 0))] * (2 * npanels),
        out_specs=pl.BlockSpec((tr, npanels * NH), lambda i: (i, 0)),
        out_shape=jax.ShapeDtypeStruct((N_PAD, npanels * NH), jnp.float32),
    )


def _att_matrices(att_src, att_dst):
    heads, ch = att_src.shape
    eye = jnp.eye(heads, 8, dtype=jnp.float32)
    a_s = (att_src[:, :, None] * eye[:, None, :]).reshape(heads * ch, 8)
    a_d = (att_dst[:, :, None] * eye[:, None, :]).reshape(heads * ch, 8)
    return a_s, a_d


def _gat_conv(h, src_p, dst_p, att_src, att_dst, bias, heads, zeros128,
              relu_out):
    a_s, a_d = _att_matrices(att_src, att_dst)
    sa = jnp.pad(_matmul(h, a_s, exact=True), ((0, 0), (0, NH - 8)))
    sd = jnp.pad(_matmul(h, a_d, exact=True), ((0, 0), (0, NH - 8)))
    ga, gb = _sc_gather2(EL_PAD, NH)(sa, sd, src_p, dst_p)
    exT = _ex_kernel(EL_PAD, EL)(ga, gb)

    tables = [h[:, p * NH:(p + 1) * NH] for p in range(heads)]
    exs = [exT[p] for p in range(heads)]
    parts = _sc_agg(EL_PAD, N, heads, True, 128)(
        *tables, *exs, src_p, dst_p, zeros128)
    den0, den1 = parts[2 * heads], parts[2 * heads + 1]
    out = _combine_kernel(heads, relu_out)(
        den0, den1, *parts[:2 * heads], bias.reshape(1, heads * NH))
    return out[:N]


def kernel(x, edge_index, params):
    src = edge_index[0].astype(jnp.int32)
    dst = edge_index[1].astype(jnp.int32)
    loop = jnp.arange(N, dtype=jnp.int32)
    padl = jnp.zeros((EL_PAD - EL,), jnp.int32)
    src_p = jnp.concatenate([src, loop, padl])
    dst_p = jnp.concatenate([dst, loop, padl])
    pade = jnp.full((E_PAD - E,), N, jnp.int32)
    row_p = jnp.concatenate([src, pade])
    col_p = jnp.concatenate([dst, pade])

    zeros128 = jnp.zeros((N_PAD, NH), jnp.float32)
    ones_e = jnp.concatenate([jnp.ones((E,), jnp.float32),
                              jnp.zeros((E_PAD - E,), jnp.float32)])

    x0 = x
    for i in range(DEPTH):
        p = params['block%d' % i]
        h1 = _matmul(x, p['W1'])
        h1a = _gat_conv(h1, src_p, dst_p, p['as1'], p['ad1'], p['b1'],
                        HEADS, zeros128, relu_out=True)
        h2 = _matmul(h1a, p['W2'])
        h2o = _gat_conv(h2, src_p, dst_p, p['as2'], p['ad2'], p['b2'],
                        1, zeros128, relu_out=False)
        skip = _matmul(x, p['Wskip'], p['bskip'])
        x = _bn_kernel()(h2o, skip, p['gamma'].reshape(1, NH),
                         p['beta'].reshape(1, NH))

    xm = jnp.concatenate([x, x0], axis=1)

    dparts = _sc_agg(E_PAD, N, 0, True, 160)(ones_e, row_p, row_p, zeros128)
    dinv128 = _dinv_kernel()(dparts[0], dparts[1])
    gr, gc = _sc_gather2(E_PAD, NH)(dinv128, dinv128, row_p, col_p)
    w = _chebw_kernel()(gr, gc)

    xm_pad = jnp.pad(xm, ((0, N_PAD - N), (0, 0)))
    xtables = [xm_pad[:, p * NH:(p + 1) * NH] for p in range(3)]
    tparts = _sc_agg(E_PAD, N_PAD, 3, False, 160)(
        *xtables, w, w, w, row_p, col_p, zeros128)
    tx1 = _sum_partials_kernel(3)(*tparts)[:N]

    pm = params['mix']
    xcat = jnp.concatenate([xm, tx1], axis=1)
    wcat = jnp.concatenate([pm['W0'], pm['W1']], axis=0)
    return _matmul(xcat, wcat, pm['b'])
